# Initial kernel scaffold; baseline (speedup 1.0000x reference)
import jax
import jax.numpy as jnp
from jax.experimental import pallas as pl


def kernel(node, edge_index, edge, hidden, params):
    raise NotImplementedError("write your pallas kernel here")


# speedup vs baseline: 3.1480x; 3.1480x over previous
import functools
import jax
import jax.numpy as jnp
from jax import lax
from jax.experimental import pallas as pl
from jax.experimental.pallas import tpu as pltpu
from jax.experimental.pallas import tpu_sc as plsc

D = 128
E = 8192
N_NODES = 10000
N_PAD = 10240

NC = 2
NS = 16
NW = NC * NS

_F32 = jnp.float32


def _dot_t(a, b):
    return lax.dot_general(a, b, (((1,), (1,)), ((), ())),
                           preferred_element_type=_F32)


def _bn_relu(y, g, b):
    mu = jnp.mean(y, axis=0, keepdims=True)
    var = jnp.mean((y - mu) * (y - mu), axis=0, keepdims=True)
    return jnp.maximum(g * ((y - mu) * lax.rsqrt(var + 1e-5)) + b, 0.0)


def _encoder_body(edge_ref, w0_ref, g0_ref, b0_ref, w1_ref, g1_ref, b1_ref,
                  w2_ref, g2_ref, b2_ref, w3_ref, g3_ref, b3_ref,
                  h3_out, w3s_out, t_out):
    x = edge_ref[...]
    h = _bn_relu(_dot_t(x, w0_ref[...]), g0_ref[...], b0_ref[...])
    h = _bn_relu(_dot_t(h, w1_ref[...]), g1_ref[...], b1_ref[...])
    h3 = _bn_relu(_dot_t(h, w2_ref[...]), g2_ref[...], b2_ref[...])
    h3_out[...] = h3
    hbar = jnp.mean(h3, axis=0, keepdims=True)
    c = lax.dot_general(h3, h3, (((0,), (0,)), ((), ())),
                        preferred_element_type=_F32) * (1.0 / E)
    w3 = w3_ref[...]
    mu3 = _dot_t(w3, hbar)
    q = jnp.sum(_dot_t(w3, c) * w3, axis=1, keepdims=True)
    var3 = q - mu3 * mu3
    s = g3_ref[...] * lax.rsqrt(var3 + 1e-5)
    t_out[...] = b3_ref[...] - mu3 * s
    w3s_out[...] = w3 * s


def _encoder(edge, p):
    out_shapes = (
        jax.ShapeDtypeStruct((E, D), _F32),
        jax.ShapeDtypeStruct((D * D, D), _F32),
        jax.ShapeDtypeStruct((D * D, 1), _F32),
    )
    return pl.pallas_call(
        _encoder_body,
        out_shape=out_shapes,
    )(edge, p['W0'], p['g0'].reshape(1, -1), p['b0'].reshape(1, -1),
      p['W1'], p['g1'].reshape(1, -1), p['b1'].reshape(1, -1),
      p['W2'], p['g2'].reshape(1, -1), p['b2'].reshape(1, -1),
      p['W3'], p['g3'].reshape(-1, 1), p['b3'].reshape(-1, 1))


TE = 256


def _msg_body(h3_ref, xi_ref, w3s_ref, tmat_ref, out_ref):
    y = _dot_t(h3_ref[...], w3s_ref[...])
    xi = xi_ref[...]
    acc = lax.dot_general(xi, tmat_ref[...], (((1,), (0,)), ((), ())),
                          preferred_element_type=_F32)
    for d in range(D):
        acc = acc + y[:, d * D:(d + 1) * D] * xi[:, d:d + 1]
    out_ref[...] = acc


def _messages(h3, xi, w3s, tmat):
    grid = (E // TE,)
    return pl.pallas_call(
        _msg_body,
        grid=grid,
        in_specs=[
            pl.BlockSpec((TE, D), lambda i: (i, 0)),
            pl.BlockSpec((TE, D), lambda i: (i, 0)),
            pl.BlockSpec((D * D, D), lambda i: (0, 0)),
            pl.BlockSpec((D, D), lambda i: (0, 0)),
        ],
        out_specs=pl.BlockSpec((TE, D), lambda i: (i, 0)),
        out_shape=jax.ShapeDtypeStruct((E, D), _F32),
    )(h3, xi, w3s, tmat)


B_PER_W = E // NW
KB_G = B_PER_W // 128


def _gather_body(node_hbm, src_hbm, out_hbm, idx_v, rows_v, sem):
    wid = lax.axis_index("s") * NC + lax.axis_index("c")
    pltpu.sync_copy(src_hbm.at[pl.ds(wid * KB_G, KB_G)], idx_v)
    for b in range(KB_G):
        pltpu.async_copy(node_hbm.at[idx_v.at[b]],
                         rows_v.at[pl.ds(b * 128, 128)], sem).wait()
    pltpu.sync_copy(rows_v, out_hbm.at[pl.ds(wid * B_PER_W, B_PER_W)])


def _gather(node, src2d):
    f = functools.partial(
        pl.kernel,
        out_type=jax.ShapeDtypeStruct((E, D), _F32),
        mesh=plsc.VectorSubcoreMesh(core_axis_name="c", subcore_axis_name="s", num_cores=NC, num_subcores=NS),
        scratch_types=[
            pltpu.VMEM((KB_G, 128), jnp.int32),
            pltpu.VMEM((B_PER_W, D), _F32),
            pltpu.SemaphoreType.DMA,
        ],
    )(_gather_body)
    return f(node, src2d)


E_PER_S = E // NS
KB_S = E_PER_S // 128
N_HALF = N_PAD // NC
T_ROWS = N_HALF + 128
R_Z = T_ROWS // NS
R_OUT = N_HALF // NS
DUMP = N_HALF


def _scatter_body(data_hbm, dst_hbm, zeros_hbm, out_hbm, idx_v, data_v,
                  acc_sh, sem):
    cid = lax.axis_index("c")
    sid = lax.axis_index("s")
    base = cid * N_HALF
    pltpu.sync_copy(zeros_hbm, acc_sh.at[pl.ds(sid * R_Z, R_Z)])
    pltpu.sync_copy(data_hbm.at[pl.ds(sid * E_PER_S, E_PER_S)], data_v)
    pltpu.sync_copy(dst_hbm.at[pl.ds(sid * KB_S, KB_S)], idx_v)
    for r in range(KB_S):
        for j in range(128 // 16):
            v = idx_v[r, pl.ds(j * 16, 16)] - base
            ok = (v >= 0) & (v < N_HALF)
            idx_v[r, pl.ds(j * 16, 16)] = jnp.where(ok, v, DUMP)
    plsc.subcore_barrier()
    for b in range(KB_S):
        pltpu.sync_copy(data_v.at[pl.ds(b * 128, 128)],
                        acc_sh.at[idx_v.at[b]], add=True)
    plsc.subcore_barrier()
    pltpu.sync_copy(acc_sh.at[pl.ds(sid * R_OUT, R_OUT)],
                    out_hbm.at[pl.ds(base + sid * R_OUT, R_OUT)])


def _scatter(data, dst2d, zeros_blk):
    f = functools.partial(
        pl.kernel,
        out_type=jax.ShapeDtypeStruct((N_PAD, D), _F32),
        mesh=plsc.VectorSubcoreMesh(core_axis_name="c", subcore_axis_name="s", num_cores=NC, num_subcores=NS),
        scratch_types=[
            pltpu.VMEM((KB_S, 128), jnp.int32),
            pltpu.VMEM((E_PER_S, D), _F32),
            pltpu.VMEM_SHARED((T_ROWS, D), _F32),
            pltpu.SemaphoreType.DMA,
        ],
    )(_scatter_body)
    return f(data, dst2d, zeros_blk)


TN = 1280


def _gru_body(sums_ref, cnt_ref, h0_ref, bias_ref, wih_ref, whh_ref,
              bih_ref, bhh_ref, out_ref):
    cnt = jnp.maximum(cnt_ref[...][:, 0:1], 1.0)
    m = jnp.maximum(sums_ref[...] / cnt + bias_ref[...], 0.0)
    h0 = h0_ref[...]
    gi = _dot_t(m, wih_ref[...]) + bih_ref[...]
    gh = _dot_t(h0, whh_ref[...]) + bhh_ref[...]
    r = jax.nn.sigmoid(gi[:, :D] + gh[:, :D])
    z = jax.nn.sigmoid(gi[:, D:2 * D] + gh[:, D:2 * D])
    n = jnp.tanh(gi[:, 2 * D:] + r * gh[:, 2 * D:])
    out_ref[...] = (1.0 - z) * n + z * h0


def _gru(sums, cnt, h0p, p):
    grid = (N_PAD // TN,)
    return pl.pallas_call(
        _gru_body,
        grid=grid,
        in_specs=[
            pl.BlockSpec((TN, D), lambda i: (i, 0)),
            pl.BlockSpec((TN, D), lambda i: (i, 0)),
            pl.BlockSpec((TN, D), lambda i: (i, 0)),
            pl.BlockSpec((1, D), lambda i: (0, 0)),
            pl.BlockSpec((3 * D, D), lambda i: (0, 0)),
            pl.BlockSpec((3 * D, D), lambda i: (0, 0)),
            pl.BlockSpec((1, 3 * D), lambda i: (0, 0)),
            pl.BlockSpec((1, 3 * D), lambda i: (0, 0)),
        ],
        out_specs=pl.BlockSpec((TN, D), lambda i: (i, 0)),
        out_shape=jax.ShapeDtypeStruct((N_PAD, D), _F32),
    )(sums, cnt, h0p, p['bias'].reshape(1, D), p['W_ih'],
      p['W_hh'], p['b_ih'].reshape(1, 3 * D), p['b_hh'].reshape(1, 3 * D))


def kernel(node, edge_index, edge, hidden, params):
    ei = edge_index.astype(jnp.int32)
    src2d = ei[:, 0].reshape(E // 128, 128)
    dst2d = ei[:, 1].reshape(E // 128, 128)

    xi = _gather(node, src2d)
    h3, w3s, t = _encoder(edge, params)
    tmat = t.reshape(D, D)
    msg = _messages(h3, xi, w3s, tmat)

    zeros_blk = jnp.zeros((R_Z, D), _F32)
    ones_all = jnp.ones((E, D), _F32)
    sums = _scatter(msg, dst2d, zeros_blk)
    cnt = _scatter(ones_all, dst2d, zeros_blk)

    h0p = jnp.pad(hidden[0], ((0, N_PAD - N_NODES), (0, 0)))
    out = _gru(sums, cnt, h0p, params)
    hnew = out[:N_NODES]
    return hnew, hnew[None, :, :]
